# baseline (speedup 1.0000x reference)
import functools

import jax
import jax.numpy as jnp
from jax import lax
from jax.experimental import pallas as pl
from jax.experimental.pallas import tpu as pltpu
from jax.experimental.pallas import tpu_sc as plsc

B, L = 4096, 200
N_TOK = B * L
N_PIECE = 6 * N_TOK
NC, NS = 2, 16
NW = NC * NS
P_PER_W = N_PIECE // NW
T_PER_W = N_TOK // NW
JT = 40
JP = 6 * JT
N_CH = P_PER_W // JP
N_PAIR = N_CH // 2


def _sc_piece_gather(idx_pairs, tp):
    mesh = plsc.VectorSubcoreMesh(core_axis_name="c", subcore_axis_name="s")

    @functools.partial(
        pl.kernel,
        out_type=jax.ShapeDtypeStruct((N_PIECE, 128), jnp.float32),
        mesh=mesh,
        scratch_types=(
            [pltpu.VMEM((2 * T_PER_W,), jnp.int32)]
            + [pltpu.VMEM((JP,), jnp.int32) for _ in range(2)]
            + [pltpu.VMEM((JP, 128), jnp.float32) for _ in range(2)]
            + [pltpu.SemaphoreType.DMA for _ in range(4)]
        ),
    )
    def k(idx_hbm, tp_hbm, out_hbm, idx_all, pb0, pb1, rows0, rows1,
          sg0, sg1, sw0, sw1):
        pb = (pb0, pb1)
        rows = (rows0, rows1)
        sem_g = (sg0, sg1)
        sem_w = (sw0, sw1)
        wid = lax.axis_index("s") * NC + lax.axis_index("c")
        base = wid * P_PER_W

        iota = lax.iota(jnp.int32, 16)
        half = iota >> 3
        perm0 = (2 * iota) & 15
        perm1 = (2 * iota + half) & 15
        perm2 = (2 * iota + 1) & 15
        off0 = half * 8
        off1 = 16 - half * 16
        off2 = 8 + half * 8

        dnums = lax.GatherDimensionNumbers(
            offset_dims=(), collapsed_slice_dims=(0,), start_index_map=(0,))

        def take16(vec, perm):
            return lax.gather(vec, perm[:, None], dnums, slice_sizes=(1,),
                              mode=lax.GatherScatterMode.PROMISE_IN_BOUNDS)

        def construct(c, v):
            toff = c * (2 * JT)
            for bk in range(JT // 8):
                iv = idx_all[pl.ds(toff + bk * 16, 16)]
                pbase = 24 * (iv >> 3) + (iv & 7)
                g0 = take16(pbase, perm0) + off0
                g1 = take16(pbase, perm1) + off1
                g2 = take16(pbase, perm2) + off2
                pb[v][pl.ds(bk * 48, 16)] = g0
                pb[v][pl.ds(bk * 48 + 16, 16)] = g1
                pb[v][pl.ds(bk * 48 + 32, 16)] = g2

        def start_gather(p, v):
            pltpu.async_copy(tp_hbm.at[pb[v]], rows[p], sem_g[p])

        def wait_gather(p):
            pltpu.make_async_copy(out_hbm.at[pl.ds(0, JP)], rows[p],
                                  sem_g[p]).wait()

        def start_write(c, p):
            pltpu.async_copy(rows[p], out_hbm.at[pl.ds(base + c * JP, JP)],
                             sem_w[p])

        def wait_write(p):
            pltpu.make_async_copy(rows[p], out_hbm.at[pl.ds(0, JP)],
                                  sem_w[p]).wait()

        pltpu.sync_copy(idx_hbm.at[pl.ds(wid * 2 * T_PER_W, 2 * T_PER_W)],
                        idx_all)
        construct(0, 0)
        start_gather(0, 0)
        construct(1, 1)

        def body(i, carry):
            c0 = 2 * i
            wait_gather(0)
            start_write(c0, 0)

            @pl.when(i > 0)
            def _():
                wait_write(1)
            start_gather(1, 1)

            @pl.when(i < N_PAIR - 1)
            def _():
                construct(c0 + 2, 0)

            wait_gather(1)
            start_write(c0 + 1, 1)

            @pl.when(i < N_PAIR - 1)
            def _():
                wait_write(0)
                start_gather(0, 0)
                construct(c0 + 3, 1)
            return carry

        lax.fori_loop(0, N_PAIR, body, 0)
        wait_write(0)
        wait_write(1)

    return k(idx_pairs, tp)


def kernel(left_idx, up_idx, table):
    n_pad = -table.shape[0] % 8
    tp = (jnp.pad(table, ((0, n_pad), (0, 0)))
          .reshape(-1, 8, 3, 128).transpose(0, 2, 1, 3).reshape(-1, 128))
    idx_pairs = jnp.stack(
        (left_idx.reshape(-1), up_idx.reshape(-1)), axis=-1).reshape(-1)
    out = _sc_piece_gather(idx_pairs, tp)
    return (out.reshape(B, 25, 6, 8, 128)
               .transpose(0, 1, 3, 2, 4)
               .reshape(B, L, 6 * 128))


# speedup vs baseline: 1.2902x; 1.2902x over previous
import functools

import jax
import jax.numpy as jnp
from jax import lax
from jax.experimental import pallas as pl
from jax.experimental.pallas import tpu as pltpu
from jax.experimental.pallas import tpu_sc as plsc

B, L = 4096, 200
D = 384
N_TOK = B * L
N_PIECE = 6 * N_TOK
NC, NS = 2, 16
NW = NC * NS
P_PER_W = N_PIECE // NW
JP = 480
N_CH = P_PER_W // JP
N_PAIR = N_CH // 2


def _sc_piece_gather(pidx, tp):
    mesh = plsc.VectorSubcoreMesh(core_axis_name="c", subcore_axis_name="s")

    @functools.partial(
        pl.kernel,
        out_type=jax.ShapeDtypeStruct((N_PIECE, 128), jnp.float32),
        mesh=mesh,
        scratch_types=(
            [pltpu.VMEM((2 * JP,), jnp.int32) for _ in range(4)]
            + [pltpu.VMEM((JP, 128), jnp.float32) for _ in range(2)]
            + [pltpu.SemaphoreType.DMA for _ in range(8)]
        ),
    )
    def k(pidx_hbm, tp_hbm, out_hbm, ib0, ib1, ib2, ib3, rows0, rows1,
          si0, si1, si2, si3, sg0, sg1, sw0, sw1):
        ib = (ib0, ib1, ib2, ib3)
        sem_i = (si0, si1, si2, si3)
        rows = (rows0, rows1)
        sem_g = (sg0, sg1)
        sem_w = (sw0, sw1)
        wid = lax.axis_index("s") * NC + lax.axis_index("c")
        base = wid * P_PER_W

        def start_iload(pair, v):
            pltpu.async_copy(
                pidx_hbm.at[pl.ds(base + pair * 2 * JP, 2 * JP)],
                ib[v], sem_i[v])

        def wait_iload(v):
            pltpu.make_async_copy(pidx_hbm.at[pl.ds(0, 2 * JP)], ib[v],
                                  sem_i[v]).wait()

        def start_gather(c, p, v, pos):
            pltpu.async_copy(
                tp_hbm.at[ib[v].at[pl.ds(pos * JP, JP)]], rows[p], sem_g[p])

        def wait_gather(p):
            pltpu.make_async_copy(out_hbm.at[pl.ds(0, JP)], rows[p],
                                  sem_g[p]).wait()

        def start_write(c, p):
            pltpu.async_copy(rows[p], out_hbm.at[pl.ds(base + c * JP, JP)],
                             sem_w[p])

        def wait_write(p):
            pltpu.make_async_copy(rows[p], out_hbm.at[pl.ds(0, JP)],
                                  sem_w[p]).wait()

        pltpu.sync_copy(pidx_hbm.at[pl.ds(base, 2 * JP)], ib0)
        start_iload(1, 1)
        start_gather(0, 0, 0, 0)

        def body(qi, carry):
            for u in range(4):
                i = 4 * qi + u
                c0 = 2 * i

                @pl.when(i < N_PAIR - 2)
                def _():
                    start_iload(i + 2, (u + 2) % 4)

                wait_gather(0)
                start_write(c0, 0)

                @pl.when(i > 0)
                def _():
                    wait_write(1)
                start_gather(c0 + 1, 1, u, 1)

                wait_gather(1)
                start_write(c0 + 1, 1)

                @pl.when(i < N_PAIR - 1)
                def _():
                    wait_write(0)
                    wait_iload((u + 1) % 4)
                    start_gather(c0 + 2, 0, (u + 1) % 4, 0)
            return carry

        lax.fori_loop(0, N_PAIR // 4, body, 0)
        wait_write(0)
        wait_write(1)

    return k(pidx, tp)


def kernel(left_idx, up_idx, table):
    n_pad = -table.shape[0] % 8
    tp = (jnp.pad(table, ((0, n_pad), (0, 0)))
          .reshape(-1, 8, 3, 128).transpose(0, 2, 1, 3).reshape(-1, 128))
    li = left_idx.reshape(B, 25, 8)
    ui = up_idx.reshape(B, 25, 8)

    def piece_base(r):
        return 24 * (r >> 3) + (r & 7)

    cb = jnp.arange(6, dtype=jnp.int32).reshape(1, 1, 6, 1)
    pidx = (jnp.where(cb < 3, piece_base(li)[:, :, None, :],
                      piece_base(ui)[:, :, None, :])
            + 8 * (cb % 3)).astype(jnp.int32).reshape(-1)

    out = _sc_piece_gather(pidx, tp)
    return (out.reshape(B, 25, 6, 8, 128)
               .transpose(0, 1, 3, 2, 4)
               .reshape(B, L, 6 * 128))
